# baseline (speedup 1.0000x reference)
import functools

import jax
import jax.numpy as jnp
from jax import lax
from jax.experimental import pallas as pl
from jax.experimental.pallas import tpu as pltpu
from jax.experimental.pallas import tpu_sc as plsc

_NU = 5000
_HALF = 5120
_NP = 2 * _HALF
_D = 128
_NC, _NS = 2, 16
_C = 80
_NCH = 125
_RPW = _HALF // _NS
_PAD = _HALF - _NU


def _propagate(table, srcb, dstb, wb):
    mesh = plsc.VectorSubcoreMesh(core_axis_name="c", subcore_axis_name="s",
                                  num_cores=_NC, num_subcores=_NS)

    @functools.partial(
        pl.kernel,
        out_type=jax.ShapeDtypeStruct((_NP, _D), jnp.float32),
        mesh=mesh,
        scratch_types=[
            pltpu.VMEM((_NCH, _C), jnp.int32),
            pltpu.VMEM((_NCH, _C), jnp.int32),
            pltpu.VMEM((_NCH, _C), jnp.float32),
            pltpu.VMEM((_C, _D), jnp.float32),
            pltpu.VMEM((_C, _D), jnp.float32),
            pltpu.VMEM((64, _D), jnp.float32),
            pltpu.VMEM_SHARED((_HALF, _D), jnp.float32),
            pltpu.SemaphoreType.DMA,
            pltpu.SemaphoreType.DMA,
        ],
    )
    def k(t_hbm, src_hbm, dst_hbm, w_hbm, out_hbm,
          src_v, dst_v, w_v, bufa, bufb, zbuf, acc, sema, semb):
        c = lax.axis_index("c")
        s = lax.axis_index("s")
        w_id = c * _NS + s

        pltpu.sync_copy(src_hbm.at[w_id], src_v)
        pltpu.sync_copy(dst_hbm.at[w_id], dst_v)
        pltpu.sync_copy(w_hbm.at[w_id], w_v)

        zero16 = jnp.zeros((16,), jnp.float32)

        def zrow(i, _):
            for kk in range(_D // 16):
                zbuf[i, pl.ds(kk * 16, 16)] = zero16
            return 0

        lax.fori_loop(0, 64, zrow, 0)
        base = s * _RPW
        for kk in range(_RPW // 64):
            pltpu.sync_copy(zbuf, acc.at[pl.ds(base + kk * 64, 64)])
        plsc.subcore_barrier()

        def gather(j, buf, sem):
            return pltpu.make_async_copy(t_hbm.at[src_v.at[j]], buf, sem)

        gather(0, bufa, sema).start()
        gather(1, bufb, semb).start()

        def process(j, buf, sem):
            gather(j, buf, sem).wait()

            def mulgroup(g, _):
                w16 = w_v[j, pl.ds(g * 16, 16)]
                for l in range(16):
                    w_s = w16[l]
                    row = g * 16 + l
                    for kk in range(_D // 16):
                        sl = pl.ds(kk * 16, 16)
                        buf[row, sl] = buf[row, sl] * w_s
                return 0

            lax.fori_loop(0, _C // 16, mulgroup, 0)
            pltpu.sync_copy(buf, acc.at[dst_v.at[j]], add=True)

        def step(j0, _):
            process(j0, bufa, sema)

            @pl.when(j0 + 2 < _NCH)
            def _():
                gather(j0 + 2, bufa, sema).start()

            process(j0 + 1, bufb, semb)

            @pl.when(j0 + 3 < _NCH)
            def _():
                gather(j0 + 3, bufb, semb).start()

            return 0

        lax.fori_loop(0, _NCH // 2, lambda t, u: step(t * 2, u), 0)
        if _NCH % 2:
            process(_NCH - 1, bufa, sema)

        plsc.subcore_barrier()
        pltpu.sync_copy(acc.at[pl.ds(base, _RPW)],
                        out_hbm.at[pl.ds(c * _HALF + base, _RPW)])

    return k(table, srcb, dstb, wb)


def kernel(user_emb, item_emb, edge_index, edge_weight):
    src = edge_index[0].astype(jnp.int32)
    dst = edge_index[1].astype(jnp.int32)
    w = edge_weight.astype(jnp.float32)
    e2 = src.shape[0] // 2

    src_r = jnp.concatenate([src[e2:], src[:e2]])
    dst_r = jnp.concatenate([dst[e2:], dst[:e2] - _NU])
    w_r = jnp.concatenate([w[e2:], w[:e2]])
    src_g = src_r + _PAD * (src_r >= _NU).astype(jnp.int32)

    nb = _NC * _NS
    srcb = src_g.reshape(nb, _NCH, _C)
    dstb = dst_r.reshape(nb, _NCH, _C)
    wb = w_r.reshape(nb, _NCH, _C)

    pad = jnp.zeros((_PAD, _D), jnp.float32)
    e0 = jnp.concatenate([user_emb, pad, item_emb, pad], axis=0)

    e1 = _propagate(e0, srcb, dstb, wb)
    e2_ = _propagate(e1, srcb, dstb, wb)
    e3 = _propagate(e2_, srcb, dstb, wb)

    user_all = jnp.stack(
        [user_emb, e1[:_NU], e2_[:_NU], e3[:_NU]], axis=1)
    item_all = jnp.stack(
        [item_emb, e1[_HALF:_HALF + _NU], e2_[_HALF:_HALF + _NU],
         e3[_HALF:_HALF + _NU]], axis=1)
    return (user_all, item_all)
